# baseline (speedup 1.0000x reference)
import functools

import jax
import jax.numpy as jnp
from jax import lax
from jax.experimental import pallas as pl
from jax.experimental.pallas import tpu as pltpu
from jax.experimental.pallas import tpu_sc as plsc

D_VOCAB = 1000000
D_MODEL = 64
B = 4096
L = 200
B_TOTAL = B * L

_info = plsc.get_sparse_core_info()
_NC, _NS = _info.num_cores, _info.num_subcores
_NW = _NC * _NS

_BT = B // 128
_UNITS = L * _BT
_PER_W = _UNITS // _NW
_NBUF = 2


@functools.partial(
    pl.kernel,
    out_type=jax.ShapeDtypeStruct((L, 8, _BT, 8, 128), jnp.float32),
    mesh=plsc.VectorSubcoreMesh(core_axis_name="c", subcore_axis_name="s"),
    compiler_params=pltpu.CompilerParams(use_tc_tiling_on_sc=False, needs_layout_passes=False),
    scratch_types=[
        pltpu.VMEM((_PER_W * 128,), jnp.int32),
        pltpu.VMEM((_NBUF, 128, 128), jnp.float32),
        pltpu.VMEM((_NBUF, 8, 8, 128), jnp.float32),
        [pltpu.SemaphoreType.DMA] * _NBUF,
        [pltpu.SemaphoreType.DMA] * _NBUF,
    ],
)
def _embed_sc(xt_hbm, w_hbm, out_hbm, idx_v, rows_v, tall_v, gsems, wsems):
    wid = lax.axis_index("s") * _NC + lax.axis_index("c")
    u_base = wid * _PER_W

    pltpu.sync_copy(xt_hbm.at[pl.ds(u_base * 128, _PER_W * 128)], idx_v)

    iota = jax.lax.iota(jnp.int32, 16)
    b_idx = [iota + bg * 16 for bg in range(8)]

    def fire(j, buf):
        pltpu.async_copy(
            w_hbm.at[idx_v.at[pl.ds(j * 128, 128)]],
            rows_v.at[buf],
            gsems[buf],
        )

    def drain_gather(buf):
        pltpu.make_async_copy(w_hbm.at[pl.ds(0, 128)], rows_v.at[buf],
                              gsems[buf]).wait()

    def wait_wb(buf):
        pltpu.make_async_copy(tall_v.at[buf], out_hbm.at[0, pl.ds(0, 8), 0],
                              wsems[buf]).wait()

    for buf in range(_NBUF):
        fire(buf, buf)

    def body(jj, carry):
        for buf in range(_NBUF):
            j = jj * _NBUF + buf
            u = u_base + j
            l = u // _BT
            bt = u % _BT

            drain_gather(buf)

            @pl.when(j >= _NBUF)
            def _():
                wait_wb(buf)

            g = rows_v.at[buf]
            for m8 in range(8):
                for mr in range(8):
                    m = jnp.full((16,), m8 * 8 + mr, dtype=jnp.int32)
                    for bg in range(8):
                        val = plsc.load_gather(g, [b_idx[bg], m])
                        tall_v[buf, m8, mr, pl.ds(bg * 16, 16)] = val

            pltpu.async_copy(tall_v.at[buf], out_hbm.at[l, pl.ds(0, 8), bt],
                             wsems[buf])

            @pl.when(j + _NBUF < _PER_W)
            def _():
                fire(j + _NBUF, buf)

        return carry

    lax.fori_loop(0, _PER_W // _NBUF, body, 0)

    for buf in range(_NBUF):
        wait_wb(buf)


def kernel(x, weight):
    w_pad = jnp.pad(weight, ((0, 0), (0, 128 - D_MODEL)))
    xt_flat = x.T.reshape(-1)
    u = _embed_sc(xt_flat, w_pad)
    return u.transpose(2, 4, 0, 1, 3).reshape(B, L, D_MODEL)


# speedup vs baseline: 1.0845x; 1.0845x over previous
import functools

import jax
import jax.numpy as jnp
from jax import lax
from jax.experimental import pallas as pl
from jax.experimental.pallas import tpu as pltpu
from jax.experimental.pallas import tpu_sc as plsc

D_VOCAB = 1000000
D_MODEL = 64
B = 4096
L = 200

_info = plsc.get_sparse_core_info()
_NC, _NS = _info.num_cores, _info.num_subcores
_NW = _NC * _NS

_BT = B // 128
_UNITS = L * _BT
_PER_W = _UNITS // _NW
_NBUF = 2


@functools.partial(
    pl.kernel,
    out_type=jax.ShapeDtypeStruct((L, 8, _BT, 1024), jnp.float32),
    mesh=plsc.VectorSubcoreMesh(core_axis_name="c", subcore_axis_name="s"),
    compiler_params=pltpu.CompilerParams(
        use_tc_tiling_on_sc=False, needs_layout_passes=False),
    scratch_types=[
        pltpu.VMEM((_PER_W * 128,), jnp.int32),
        pltpu.VMEM((_NBUF, 128, 128), jnp.float32),
        pltpu.VMEM((_NBUF, 8, 1024), jnp.float32),
        [pltpu.SemaphoreType.DMA] * _NBUF,
        [pltpu.SemaphoreType.DMA] * _NBUF,
    ],
)
def _embed_sc(xt_hbm, w_hbm, out_hbm, idx_v, rows_v, tall_v, gsems, wsems):
    wid = lax.axis_index("s") * _NC + lax.axis_index("c")
    u_base = wid * _PER_W

    pltpu.sync_copy(xt_hbm.at[pl.ds(u_base * 128, _PER_W * 128)], idx_v)

    iota = jax.lax.iota(jnp.int32, 16)
    b_idx = [iota + bg * 16 for bg in range(8)]

    def fire(j, buf):
        pltpu.async_copy(
            w_hbm.at[idx_v.at[pl.ds(j * 128, 128)]],
            rows_v.at[buf],
            gsems[buf],
        )

    def drain_gather(buf):
        pltpu.make_async_copy(w_hbm.at[pl.ds(0, 128)], rows_v.at[buf],
                              gsems[buf]).wait()

    def wait_wb(buf):
        pltpu.make_async_copy(tall_v.at[buf], out_hbm.at[0, pl.ds(0, 8), 0],
                              wsems[buf]).wait()

    for buf in range(_NBUF):
        fire(buf, buf)

    def body(jj, carry):
        for buf in range(_NBUF):
            j = jj * _NBUF + buf
            u = u_base + j
            l = u // _BT
            bt = u % _BT

            drain_gather(buf)

            @pl.when(j >= _NBUF)
            def _():
                wait_wb(buf)

            g = rows_v.at[buf]

            def mr_body(mr, c2):
                for m8 in range(8):
                    m = jnp.full((16,), m8 * 8, jnp.int32) + mr
                    for bg in range(8):
                        val = plsc.load_gather(g, [b_idx[bg], m])
                        tall_v[buf, m8, pl.ds(mr * 128 + bg * 16, 16)] = val
                return c2

            lax.fori_loop(0, 8, mr_body, 0)

            pltpu.async_copy(tall_v.at[buf], out_hbm.at[l, pl.ds(0, 8), bt],
                             wsems[buf])

            @pl.when(j + _NBUF < _PER_W)
            def _():
                fire(j + _NBUF, buf)

        return carry

    lax.fori_loop(0, _PER_W // _NBUF, body, 0)

    for buf in range(_NBUF):
        wait_wb(buf)


def kernel(x, weight):
    w_pad = jnp.pad(weight, ((0, 0), (0, 128 - D_MODEL)))
    xt_flat = x.T.reshape(-1)
    u = _embed_sc(xt_flat, w_pad)
    u5 = u.reshape(L, 8, _BT, 8, 128)
    return u5.transpose(2, 4, 0, 1, 3).reshape(B, L, D_MODEL)


# speedup vs baseline: 2.1158x; 1.9510x over previous
import functools

import jax
import jax.numpy as jnp
from jax import lax
from jax.experimental import pallas as pl
from jax.experimental.pallas import tpu as pltpu
from jax.experimental.pallas import tpu_sc as plsc

D_VOCAB = 1000000
D_MODEL = 64
B = 4096
L = 200

_info = plsc.get_sparse_core_info()
_NC, _NS = _info.num_cores, _info.num_subcores
_NW = _NC * _NS

_BT = B // 128
_UNITS = L * _BT
_PER_W = _UNITS // _NW
_NBUF = 2
_TSTRIDE = 129


@functools.partial(
    pl.kernel,
    out_type=jax.ShapeDtypeStruct((L, 8, _BT, 8, 128), jnp.float32),
    mesh=plsc.VectorSubcoreMesh(core_axis_name="c", subcore_axis_name="s"),
    compiler_params=pltpu.CompilerParams(
        use_tc_tiling_on_sc=False, needs_layout_passes=False),
    scratch_types=[
        pltpu.VMEM((_PER_W * 128,), jnp.int32),
        pltpu.VMEM((_NBUF, 128, 128), jnp.float32),
        pltpu.VMEM((_NBUF, 64, _TSTRIDE), jnp.float32),
        [pltpu.SemaphoreType.DMA] * _NBUF,
        [pltpu.SemaphoreType.DMA] * _NBUF,
    ],
)
def _embed_sc(xt_hbm, w_hbm, out_hbm, idx_v, rows_v, tall_v, gsems, wsems):
    wid = lax.axis_index("s") * _NC + lax.axis_index("c")
    u_base = wid * _PER_W

    pltpu.sync_copy(xt_hbm.at[pl.ds(u_base * 128, _PER_W * 128)], idx_v)

    iota = jax.lax.iota(jnp.int32, 16)
    m_idx = [iota + m16 * 16 for m16 in range(4)]
    zeros = jnp.zeros((16,), jnp.int32)

    def fire(j, buf):
        pltpu.async_copy(
            w_hbm.at[idx_v.at[pl.ds(j * 128, 128)]],
            rows_v.at[buf],
            gsems[buf],
        )

    def drain_gather(buf):
        pltpu.make_async_copy(w_hbm.at[pl.ds(0, 128)], rows_v.at[buf],
                              gsems[buf]).wait()

    def wait_wb(buf):
        pltpu.make_async_copy(rows_v.at[buf, pl.ds(0, 8)],
                              out_hbm.at[0, 0, 0], wsems[buf]).wait()

    for buf in range(_NBUF):
        fire(buf, buf)

    def body(jj, carry):
        for buf in range(_NBUF):
            j = jj * _NBUF + buf
            u = u_base + j
            l = u // _BT
            bt = u % _BT

            drain_gather(buf)

            @pl.when(j >= _NBUF)
            def _():
                for m8 in range(8):
                    wait_wb(buf)

            g = rows_v.at[buf]
            t = tall_v.at[buf]

            def b_body(b4, c2):
                for db in range(4):
                    b = b4 * 4 + db
                    bs = zeros + b
                    for m16 in range(4):
                        val = g[b, pl.ds(m16 * 16, 16)]
                        plsc.store_scatter(t, [m_idx[m16], bs], val)
                return c2

            lax.fori_loop(0, 32, b_body, 0)

            for m8 in range(8):
                pltpu.async_copy(
                    tall_v.at[buf, pl.ds(m8 * 8, 8), pl.ds(0, 128)],
                    out_hbm.at[l, m8, bt],
                    wsems[buf],
                )

            @pl.when(j + _NBUF < _PER_W)
            def _():
                fire(j + _NBUF, buf)

        return carry

    lax.fori_loop(0, _PER_W // _NBUF, body, 0)

    for buf in range(_NBUF):
        for m8 in range(8):
            wait_wb(buf)


def kernel(x, weight):
    w_pad = jnp.pad(weight, ((0, 0), (0, 128 - D_MODEL)))
    xt_flat = x.T.reshape(-1)
    u = _embed_sc(xt_flat, w_pad)
    return u.transpose(2, 4, 0, 1, 3).reshape(B, L, D_MODEL)


# speedup vs baseline: 2.5005x; 1.1818x over previous
import functools

import jax
import jax.numpy as jnp
from jax import lax
from jax.experimental import pallas as pl
from jax.experimental.pallas import tpu as pltpu
from jax.experimental.pallas import tpu_sc as plsc

D_VOCAB = 1000000
D_MODEL = 64
B = 4096
L = 200

_info = plsc.get_sparse_core_info()
_NC, _NS = _info.num_cores, _info.num_subcores
_NW = _NC * _NS

_BT = B // 128
_UNITS = L * _BT
_PER_W = _UNITS // _NW
_NBUF = 2
_TSTRIDE = 129


@functools.partial(
    pl.kernel,
    out_type=jax.ShapeDtypeStruct((L, 8, _BT, 8, 128), jnp.float32),
    mesh=plsc.VectorSubcoreMesh(core_axis_name="c", subcore_axis_name="s"),
    compiler_params=pltpu.CompilerParams(
        use_tc_tiling_on_sc=False, needs_layout_passes=False),
    scratch_types=[
        pltpu.VMEM((_PER_W * 128,), jnp.int32),
        pltpu.VMEM((_NBUF, 128, 128), jnp.float32),
        pltpu.VMEM((_NBUF, 64, _TSTRIDE), jnp.float32),
        [pltpu.SemaphoreType.DMA] * _NBUF,
        [pltpu.SemaphoreType.DMA] * _NBUF,
    ],
)
def _embed_sc(xt_hbm, w_hbm, out_hbm, idx_v, rows_v, tall_v, gsems, wsems):
    wid = lax.axis_index("s") * _NC + lax.axis_index("c")
    u_base = wid * _PER_W

    pltpu.sync_copy(xt_hbm.at[pl.ds(u_base * 128, _PER_W * 128)], idx_v)

    iota = jax.lax.iota(jnp.int32, 16)
    m_idx = [iota + m16 * 16 for m16 in range(4)]
    zeros = jnp.zeros((16,), jnp.int32)

    def fire(j, buf):
        pltpu.async_copy(
            w_hbm.at[idx_v.at[pl.ds(j * 128, 128)]],
            rows_v.at[buf],
            gsems[buf],
        )

    def drain_gather(buf):
        pltpu.make_async_copy(w_hbm.at[pl.ds(0, 128)], rows_v.at[buf],
                              gsems[buf]).wait()

    def wait_wb(buf):
        pltpu.make_async_copy(rows_v.at[buf, pl.ds(0, 8)],
                              out_hbm.at[0, 0, 0], wsems[buf]).wait()

    for buf in range(_NBUF):
        fire(buf, buf)

    def body(jj, carry):
        for buf in range(_NBUF):
            j = jj * _NBUF + buf
            u = u_base + j
            l = u // _BT
            bt = u % _BT

            drain_gather(buf)

            @pl.when(j >= _NBUF)
            def _():
                for m8 in range(8):
                    wait_wb(buf)

            g = rows_v.at[buf]
            t = tall_v.at[buf]

            @plsc.parallel_loop(0, 128, unroll=8)
            def b_body(b):
                bs = zeros + b
                for m16 in range(4):
                    val = g[b, pl.ds(m16 * 16, 16)]
                    plsc.store_scatter(t, [m_idx[m16], bs], val)

            for m8 in range(8):
                pltpu.async_copy(
                    tall_v.at[buf, pl.ds(m8 * 8, 8), pl.ds(0, 128)],
                    out_hbm.at[l, m8, bt],
                    wsems[buf],
                )

            @pl.when(j + _NBUF < _PER_W)
            def _():
                fire(j + _NBUF, buf)

        return carry

    lax.fori_loop(0, _PER_W // _NBUF, body, 0)

    for buf in range(_NBUF):
        for m8 in range(8):
            wait_wb(buf)


def kernel(x, weight):
    w_pad = jnp.pad(weight, ((0, 0), (0, 128 - D_MODEL)))
    xt_flat = x.T.reshape(-1)
    u = _embed_sc(xt_flat, w_pad)
    return u.transpose(2, 4, 0, 1, 3).reshape(B, L, D_MODEL)


# speedup vs baseline: 2.5810x; 1.0322x over previous
import functools

import jax
import jax.numpy as jnp
from jax import lax
from jax.experimental import pallas as pl
from jax.experimental.pallas import tpu as pltpu
from jax.experimental.pallas import tpu_sc as plsc

D_VOCAB = 1000000
D_MODEL = 64
B = 4096
L = 200

_info = plsc.get_sparse_core_info()
_NC, _NS = _info.num_cores, _info.num_subcores
_NW = _NC * _NS

_BT = B // 128
_UNITS = L * _BT
_PER_W = _UNITS // _NW
_NBUF = 4
_TSTRIDE = 129
_WPAD = 128


@functools.partial(
    pl.kernel,
    out_type=jax.ShapeDtypeStruct((L, 8, _BT, 8, 128), jnp.float32),
    mesh=plsc.VectorSubcoreMesh(core_axis_name="c", subcore_axis_name="s"),
    compiler_params=pltpu.CompilerParams(
        use_tc_tiling_on_sc=False, needs_layout_passes=False),
    scratch_types=[
        pltpu.VMEM((_PER_W * 128,), jnp.int32),
        pltpu.VMEM((_NBUF, 128, _WPAD), jnp.float32),
        pltpu.VMEM((_NBUF, 64, _TSTRIDE), jnp.float32),
        [pltpu.SemaphoreType.DMA] * _NBUF,
        [pltpu.SemaphoreType.DMA] * _NBUF,
    ],
)
def _embed_sc(xt_hbm, w_hbm, out_hbm, idx_v, rows_v, tall_v, gsems, wsems):
    wid = lax.axis_index("s") * _NC + lax.axis_index("c")
    u_base = wid * _PER_W

    pltpu.sync_copy(xt_hbm.at[pl.ds(u_base * 128, _PER_W * 128)], idx_v)

    iota = jax.lax.iota(jnp.int32, 16)
    m_idx = [iota + m16 * 16 for m16 in range(4)]
    zeros = jnp.zeros((16,), jnp.int32)

    def fire(j, buf):
        pltpu.async_copy(
            w_hbm.at[idx_v.at[pl.ds(j * 128, 128)]],
            rows_v.at[buf],
            gsems[buf],
        )

    def drain_gather(buf):
        pltpu.make_async_copy(w_hbm.at[pl.ds(0, 128)], rows_v.at[buf],
                              gsems[buf]).wait()

    def wait_wb(buf):
        pltpu.make_async_copy(rows_v.at[buf, pl.ds(0, 8)],
                              out_hbm.at[0, 0, 0], wsems[buf]).wait()

    for buf in range(_NBUF):
        fire(buf, buf)

    def body(jj, carry):
        for buf in range(_NBUF):
            j = jj * _NBUF + buf
            u = u_base + j
            l = u // _BT
            bt = u % _BT

            drain_gather(buf)

            @pl.when(j >= _NBUF)
            def _():
                for m8 in range(8):
                    wait_wb(buf)

            g = rows_v.at[buf]
            t = tall_v.at[buf]

            @plsc.parallel_loop(0, 128, unroll=8)
            def b_body(b):
                bs = zeros + b
                for m16 in range(4):
                    val = g[b, pl.ds(m16 * 16, 16)]
                    plsc.store_scatter(t, [m_idx[m16], bs], val)

            for m8 in range(8):
                pltpu.async_copy(
                    tall_v.at[buf, pl.ds(m8 * 8, 8), pl.ds(0, 128)],
                    out_hbm.at[l, m8, bt],
                    wsems[buf],
                )

            @pl.when(j + _NBUF < _PER_W)
            def _():
                fire(j + _NBUF, buf)

        return carry

    lax.fori_loop(0, _PER_W // _NBUF, body, 0)

    for buf in range(_NBUF):
        for m8 in range(8):
            wait_wb(buf)


def kernel(x, weight):
    w_pad = jnp.pad(weight, ((0, 0), (0, _WPAD - D_MODEL)))
    xt_flat = x.T.reshape(-1)
    u = _embed_sc(xt_flat, w_pad)
    return u.transpose(2, 4, 0, 1, 3).reshape(B, L, D_MODEL)
